# baseline (speedup 1.0000x reference)
import jax
import jax.numpy as jnp
from jax import lax
from jax.experimental import pallas as pl
from jax.experimental.pallas import tpu as pltpu
from jax.experimental.pallas import tpu_sc as plsc

N = 10000
E = 320000
D = 128
H = 128

NC = 2
NS = 16
NW = NC * NS
L = 16

EPW = E // NW
CB = 80
NCH = EPW // CB
NPAIR = (NCH - 1) // 2

BN = 2000
BE = 2000

_SC_MESH = dict(core_axis_name="c", subcore_axis_name="s")


def _node_tables_body(h_ref, p_ref, wr_ref, wc_ref, w3_ref, be1_ref, a_ref, b_ref):
    h = h_ref[...]
    p = p_ref[...]
    sr = jnp.sum(p * p, axis=1, keepdims=True)
    w3 = w3_ref[...]
    base = sr * w3
    a_ref[...] = jnp.dot(h, wr_ref[...], preferred_element_type=jnp.float32) \
        + base + be1_ref[...]
    b_ref[...] = jnp.dot(h, wc_ref[...], preferred_element_type=jnp.float32) + base


def _node_tables(h, positions, Wr, Wc, w3row, be1row):
    return pl.pallas_call(
        _node_tables_body,
        grid=(N // BN,),
        in_specs=[
            pl.BlockSpec((BN, D), lambda i: (i, 0)),
            pl.BlockSpec((BN, 3), lambda i: (i, 0)),
            pl.BlockSpec((D, H), lambda i: (0, 0)),
            pl.BlockSpec((D, H), lambda i: (0, 0)),
            pl.BlockSpec((1, H), lambda i: (0, 0)),
            pl.BlockSpec((1, H), lambda i: (0, 0)),
        ],
        out_specs=[
            pl.BlockSpec((BN, H), lambda i: (i, 0)),
            pl.BlockSpec((BN, H), lambda i: (i, 0)),
        ],
        out_shape=[
            jax.ShapeDtypeStruct((N, H), jnp.float32),
            jax.ShapeDtypeStruct((N, H), jnp.float32),
        ],
    )(h, positions, Wr, Wc, w3row, be1row)


def _edge_pre_body(a_hbm, b_hbm, row3_hbm, col3_hbm, px_hbm, py_hbm, pz_hbm,
                   w3_hbm, out_hbm,
                   idxR2, idxC2, bufA0, bufB0, bufA1, bufB1, bufO,
                   radv, pxv, pyv, pzv, w3v,
                   semA0, semB0, semA1, semB1, semO):
    wid = lax.axis_index("s") * NC + lax.axis_index("c")
    base = wid * EPW

    pltpu.sync_copy(row3_hbm.at[wid], idxR2)
    pltpu.sync_copy(col3_hbm.at[wid], idxC2)
    pltpu.sync_copy(px_hbm, pxv)
    pltpu.sync_copy(py_hbm, pyv)
    pltpu.sync_copy(pz_hbm, pzv)
    pltpu.sync_copy(w3_hbm, w3v)

    w3g = [w3v[pl.ds(g * L, L)] for g in range(H // L)]

    def issue(ci, bA, bB, sA, sB):
        pltpu.async_copy(a_hbm.at[idxR2.at[ci]], bA, sA)
        pltpu.async_copy(b_hbm.at[idxC2.at[ci]], bB, sB)

    def wait_gather(bA, bB, sA, sB):
        pltpu.make_async_copy(a_hbm.at[idxR2.at[0]], bA, sA).wait()
        pltpu.make_async_copy(b_hbm.at[idxC2.at[0]], bB, sB).wait()

    def compute(ci, bA, bB):
        for g in range(CB // L):
            sl = pl.ds(g * L, L)
            ir = idxR2[ci, sl]
            ic = idxC2[ci, sl]
            xr = plsc.load_gather(pxv, [ir])
            xc = plsc.load_gather(pxv, [ic])
            yr = plsc.load_gather(pyv, [ir])
            yc = plsc.load_gather(pyv, [ic])
            zr = plsc.load_gather(pzv, [ir])
            zc = plsc.load_gather(pzv, [ic])
            radv[sl] = -2.0 * (xr * xc + yr * yc + zr * zc)

        @pl.when(ci > 0)
        def _():
            pltpu.make_async_copy(bufO, out_hbm.at[pl.ds(base, CB)], semO).wait()

        def edge(e, c2):
            r = radv[pl.ds(e, L)][0]
            for g in range(H // L):
                sl = pl.ds(g * L, L)
                bufO[e, sl] = jnp.maximum(bA[e, sl] + bB[e, sl] + r * w3g[g], 0.0)
            return c2

        lax.fori_loop(0, CB, edge, 0)
        pltpu.async_copy(bufO, out_hbm.at[pl.ds(base + ci * CB, CB)], semO)

    issue(0, bufA0, bufB0, semA0, semB0)

    def pair(k, carry):
        c0 = 2 * k
        issue(c0 + 1, bufA1, bufB1, semA1, semB1)
        wait_gather(bufA0, bufB0, semA0, semB0)
        compute(c0, bufA0, bufB0)
        issue(c0 + 2, bufA0, bufB0, semA0, semB0)
        wait_gather(bufA1, bufB1, semA1, semB1)
        compute(c0 + 1, bufA1, bufB1)
        return carry

    lax.fori_loop(0, NPAIR, pair, 0)
    wait_gather(bufA0, bufB0, semA0, semB0)
    compute(NCH - 1, bufA0, bufB0)
    pltpu.make_async_copy(bufO, out_hbm.at[pl.ds(base, CB)], semO).wait()


def _edge_pre(A, B, row3, col3, px, py, pz, w3):
    return pl.kernel(
        _edge_pre_body,
        out_type=jax.ShapeDtypeStruct((E, H), jnp.float32),
        mesh=plsc.VectorSubcoreMesh(**_SC_MESH),
        compiler_params=pltpu.CompilerParams(needs_layout_passes=False),
        scratch_types=[
            pltpu.VMEM((NCH, CB), jnp.int32),
            pltpu.VMEM((NCH, CB), jnp.int32),
            pltpu.VMEM((CB, H), jnp.float32),
            pltpu.VMEM((CB, H), jnp.float32),
            pltpu.VMEM((CB, H), jnp.float32),
            pltpu.VMEM((CB, H), jnp.float32),
            pltpu.VMEM((CB, H), jnp.float32),
            pltpu.VMEM((CB + L,), jnp.float32),
            pltpu.VMEM((N,), jnp.float32),
            pltpu.VMEM((N,), jnp.float32),
            pltpu.VMEM((N,), jnp.float32),
            pltpu.VMEM((H,), jnp.float32),
            pltpu.SemaphoreType.DMA,
            pltpu.SemaphoreType.DMA,
            pltpu.SemaphoreType.DMA,
            pltpu.SemaphoreType.DMA,
            pltpu.SemaphoreType.DMA,
        ],
    )(A, B, row3, col3, px, py, pz, w3)


def _edge_mlp_body(pre_ref, w2_ref, be2_ref, f_ref):
    pre = pre_ref[...].astype(jnp.bfloat16)
    f_ref[...] = jnp.maximum(
        jnp.dot(pre, w2_ref[...].astype(jnp.bfloat16),
                preferred_element_type=jnp.float32)
        + be2_ref[...], 0.0)


def _edge_mlp(pre, We2, be2row):
    return pl.pallas_call(
        _edge_mlp_body,
        grid=(E // BE,),
        in_specs=[
            pl.BlockSpec((BE, H), lambda i: (i, 0)),
            pl.BlockSpec((H, H), lambda i: (0, 0)),
            pl.BlockSpec((1, H), lambda i: (0, 0)),
        ],
        out_specs=pl.BlockSpec((BE, H), lambda i: (i, 0)),
        out_shape=jax.ShapeDtypeStruct((E, H), jnp.float32),
    )(pre, We2, be2row)


ZR = 125
RPT = N // NS


def _scatter_body(f_hbm, row_hbm, out_hbm, idxb0, idxb1, fbuf0, fbuf1, zbuf,
                  aggsh, semI0, semI1, semL0, semL1):
    c = lax.axis_index("c")
    s = lax.axis_index("s")
    wid = s * NC + c
    base = wid * EPW

    def issue_load(ci, ib, fb, sI, sL):
        off = base + ci * CB
        pltpu.async_copy(row_hbm.at[pl.ds(off, CB)], ib, sI)
        pltpu.async_copy(f_hbm.at[pl.ds(off, CB)], fb, sL)

    def wait_load(ib, fb, sI, sL):
        pltpu.make_async_copy(row_hbm.at[pl.ds(base, CB)], ib, sI).wait()
        pltpu.make_async_copy(f_hbm.at[pl.ds(base, CB)], fb, sL).wait()

    def do_scatter(ib, fb):
        pltpu.sync_copy(fb, aggsh.at[ib], add=True)

    issue_load(0, idxb0, fbuf0, semI0, semL0)

    z16 = jnp.zeros((L,), jnp.float32)

    def zrow(i, carry):
        for g in range(H // L):
            zbuf[i, pl.ds(g * L, L)] = z16
        return carry

    lax.fori_loop(0, ZR, zrow, 0)
    for k in range(RPT // ZR):
        pltpu.sync_copy(zbuf, aggsh.at[pl.ds(s * RPT + k * ZR, ZR)])
    plsc.subcore_barrier()

    def pair(k, carry):
        c0 = 2 * k
        issue_load(c0 + 1, idxb1, fbuf1, semI1, semL1)
        wait_load(idxb0, fbuf0, semI0, semL0)
        do_scatter(idxb0, fbuf0)
        issue_load(c0 + 2, idxb0, fbuf0, semI0, semL0)
        wait_load(idxb1, fbuf1, semI1, semL1)
        do_scatter(idxb1, fbuf1)
        return carry

    lax.fori_loop(0, NPAIR, pair, 0)
    wait_load(idxb0, fbuf0, semI0, semL0)
    do_scatter(idxb0, fbuf0)

    plsc.subcore_barrier()
    pltpu.sync_copy(aggsh.at[pl.ds(s * RPT, RPT)], out_hbm.at[c, s])


def _scatter(f, row):
    return pl.kernel(
        _scatter_body,
        out_type=jax.ShapeDtypeStruct((NC, NS, RPT, H), jnp.float32),
        mesh=plsc.VectorSubcoreMesh(**_SC_MESH),
        scratch_types=[
            pltpu.VMEM((CB,), jnp.int32),
            pltpu.VMEM((CB,), jnp.int32),
            pltpu.VMEM((CB, H), jnp.float32),
            pltpu.VMEM((CB, H), jnp.float32),
            pltpu.VMEM((ZR, H), jnp.float32),
            pltpu.VMEM_SHARED((N, H), jnp.float32),
            pltpu.SemaphoreType.DMA,
            pltpu.SemaphoreType.DMA,
            pltpu.SemaphoreType.DMA,
            pltpu.SemaphoreType.DMA,
        ],
    )(f, row)


BN5 = 5000
TPB = BN5 // RPT


def _node_mlp_body(h_ref, agg_ref, wn1h_ref, wn1a_ref, bn1_ref, wn2_ref, bn2_ref, o_ref):
    hh = h_ref[...]
    agg = (agg_ref[0] + agg_ref[1]).reshape(BN5, H)
    t = jnp.maximum(
        jnp.dot(hh, wn1h_ref[...], preferred_element_type=jnp.float32)
        + jnp.dot(agg, wn1a_ref[...], preferred_element_type=jnp.float32)
        + bn1_ref[...], 0.0)
    o_ref[...] = hh + jnp.dot(t, wn2_ref[...], preferred_element_type=jnp.float32) \
        + bn2_ref[...]


def _node_mlp(h, aggp, Wn1h, Wn1a, bn1row, Wn2, bn2row):
    return pl.pallas_call(
        _node_mlp_body,
        grid=(N // BN5,),
        in_specs=[
            pl.BlockSpec((BN5, D), lambda i: (i, 0)),
            pl.BlockSpec((NC, TPB, RPT, H), lambda i: (0, i, 0, 0)),
            pl.BlockSpec((D, H), lambda i: (0, 0)),
            pl.BlockSpec((H, H), lambda i: (0, 0)),
            pl.BlockSpec((1, H), lambda i: (0, 0)),
            pl.BlockSpec((H, D), lambda i: (0, 0)),
            pl.BlockSpec((1, D), lambda i: (0, 0)),
        ],
        out_specs=pl.BlockSpec((BN5, D), lambda i: (i, 0)),
        out_shape=jax.ShapeDtypeStruct((N, D), jnp.float32),
    )(h, aggp, Wn1h, Wn1a, bn1row, Wn2, bn2row)


def kernel(h, positions, edge_index, We1, be1, We2, be2, Wn1, bn1, Wn2, bn2):
    row = edge_index[0]
    col = edge_index[1]
    row3 = row.reshape(NW, NCH, CB)
    col3 = col.reshape(NW, NCH, CB)
    pT = positions.T
    px, py, pz = pT[0], pT[1], pT[2]
    Wr = We1[:D]
    Wc = We1[D:2 * D]
    w3 = We1[2 * D]

    A, B = _node_tables(h, positions, Wr, Wc, w3.reshape(1, H), be1.reshape(1, H))
    pre = _edge_pre(A, B, row3, col3, px, py, pz, w3)
    f = _edge_mlp(pre, We2, be2.reshape(1, H))
    aggp = _scatter(f, row)
    h_out = _node_mlp(h, aggp, Wn1[:D], Wn1[D:], bn1.reshape(1, H), Wn2,
                      bn2.reshape(1, D))
    return (h_out, positions)


# speedup vs baseline: 1.1648x; 1.1648x over previous
import functools

import jax
import jax.numpy as jnp
from jax import lax
from jax.experimental import pallas as pl
from jax.experimental.pallas import tpu as pltpu
from jax.experimental.pallas import tpu_sc as plsc

N = 10000
E = 320000
D = 128
H = 128

NC = 2
NS = 16
NW = NC * NS
L = 16

EPW = E // NW
CB = 80
NCH = EPW // CB
NCHA = 63
NCHB = NCH - NCHA

BN = 2000
BE = 2000

_SC_MESH = dict(core_axis_name="c", subcore_axis_name="s")


def _node_tables_body(h_ref, p_ref, wr_ref, wc_ref, w3_ref, be1_ref, a_ref, b_ref):
    h = h_ref[...]
    p = p_ref[...]
    sr = jnp.sum(p * p, axis=1, keepdims=True)
    w3 = w3_ref[...]
    base = sr * w3
    a_ref[...] = jnp.dot(h, wr_ref[...], preferred_element_type=jnp.float32) \
        + base + be1_ref[...]
    b_ref[...] = jnp.dot(h, wc_ref[...], preferred_element_type=jnp.float32) + base


def _node_tables(h, positions, Wr, Wc, w3row, be1row):
    return pl.pallas_call(
        _node_tables_body,
        grid=(N // BN,),
        in_specs=[
            pl.BlockSpec((BN, D), lambda i: (i, 0)),
            pl.BlockSpec((BN, 3), lambda i: (i, 0)),
            pl.BlockSpec((D, H), lambda i: (0, 0)),
            pl.BlockSpec((D, H), lambda i: (0, 0)),
            pl.BlockSpec((1, H), lambda i: (0, 0)),
            pl.BlockSpec((1, H), lambda i: (0, 0)),
        ],
        out_specs=[
            pl.BlockSpec((BN, H), lambda i: (i, 0)),
            pl.BlockSpec((BN, H), lambda i: (i, 0)),
        ],
        out_shape=[
            jax.ShapeDtypeStruct((N, H), jnp.float32),
            jax.ShapeDtypeStruct((N, H), jnp.float32),
        ],
    )(h, positions, Wr, Wc, w3row, be1row)


def _edge_pre_body(nch, a_hbm, b_hbm, row3_hbm, col3_hbm, px_hbm, py_hbm,
                   pz_hbm, w3_hbm, out_hbm,
                   idxR2, idxC2, bufA0, bufB0, bufA1, bufB1, bufO,
                   radv, pxv, pyv, pzv, w3v,
                   semA0, semB0, semA1, semB1, semO):
    wid = lax.axis_index("s") * NC + lax.axis_index("c")
    base = wid * (nch * CB)

    pltpu.sync_copy(row3_hbm.at[wid], idxR2)
    pltpu.sync_copy(col3_hbm.at[wid], idxC2)
    pltpu.sync_copy(px_hbm, pxv)
    pltpu.sync_copy(py_hbm, pyv)
    pltpu.sync_copy(pz_hbm, pzv)
    pltpu.sync_copy(w3_hbm, w3v)

    w3g = [w3v[pl.ds(g * L, L)] for g in range(H // L)]

    def issue(ci, bA, bB, sA, sB):
        pltpu.async_copy(a_hbm.at[idxR2.at[ci]], bA, sA)
        pltpu.async_copy(b_hbm.at[idxC2.at[ci]], bB, sB)

    def wait_gather(bA, bB, sA, sB):
        pltpu.make_async_copy(a_hbm.at[idxR2.at[0]], bA, sA).wait()
        pltpu.make_async_copy(b_hbm.at[idxC2.at[0]], bB, sB).wait()

    def compute(ci, bA, bB):
        for g in range(CB // L):
            sl = pl.ds(g * L, L)
            ir = idxR2[ci, sl]
            ic = idxC2[ci, sl]
            xr = plsc.load_gather(pxv, [ir])
            xc = plsc.load_gather(pxv, [ic])
            yr = plsc.load_gather(pyv, [ir])
            yc = plsc.load_gather(pyv, [ic])
            zr = plsc.load_gather(pzv, [ir])
            zc = plsc.load_gather(pzv, [ic])
            radv[sl] = -2.0 * (xr * xc + yr * yc + zr * zc)

        @pl.when(ci > 0)
        def _():
            pltpu.make_async_copy(bufO, out_hbm.at[pl.ds(base, CB)], semO).wait()

        def edge(e, c2):
            r = radv[pl.ds(e, L)][0]
            for g in range(H // L):
                sl = pl.ds(g * L, L)
                bufO[e, sl] = jnp.maximum(bA[e, sl] + bB[e, sl] + r * w3g[g], 0.0)
            return c2

        lax.fori_loop(0, CB, edge, 0)
        pltpu.async_copy(bufO, out_hbm.at[pl.ds(base + ci * CB, CB)], semO)

    issue(0, bufA0, bufB0, semA0, semB0)

    def pair(k, carry):
        c0 = 2 * k
        issue(c0 + 1, bufA1, bufB1, semA1, semB1)
        wait_gather(bufA0, bufB0, semA0, semB0)
        compute(c0, bufA0, bufB0)
        issue(c0 + 2, bufA0, bufB0, semA0, semB0)
        wait_gather(bufA1, bufB1, semA1, semB1)
        compute(c0 + 1, bufA1, bufB1)
        return carry

    if nch % 2 == 1:
        lax.fori_loop(0, (nch - 1) // 2, pair, 0)
        wait_gather(bufA0, bufB0, semA0, semB0)
        compute(nch - 1, bufA0, bufB0)
    else:
        lax.fori_loop(0, (nch - 2) // 2, pair, 0)
        issue(nch - 1, bufA1, bufB1, semA1, semB1)
        wait_gather(bufA0, bufB0, semA0, semB0)
        compute(nch - 2, bufA0, bufB0)
        wait_gather(bufA1, bufB1, semA1, semB1)
        compute(nch - 1, bufA1, bufB1)
    pltpu.make_async_copy(bufO, out_hbm.at[pl.ds(base, CB)], semO).wait()


def _edge_pre(A, B, row3, col3, px, py, pz, w3, nch):
    return pl.kernel(
        functools.partial(_edge_pre_body, nch),
        out_type=jax.ShapeDtypeStruct((NW * nch * CB, H), jnp.float32),
        mesh=plsc.VectorSubcoreMesh(**_SC_MESH),
        compiler_params=pltpu.CompilerParams(needs_layout_passes=False),
        scratch_types=[
            pltpu.VMEM((nch, CB), jnp.int32),
            pltpu.VMEM((nch, CB), jnp.int32),
            pltpu.VMEM((CB, H), jnp.float32),
            pltpu.VMEM((CB, H), jnp.float32),
            pltpu.VMEM((CB, H), jnp.float32),
            pltpu.VMEM((CB, H), jnp.float32),
            pltpu.VMEM((CB, H), jnp.float32),
            pltpu.VMEM((CB + L,), jnp.float32),
            pltpu.VMEM((N,), jnp.float32),
            pltpu.VMEM((N,), jnp.float32),
            pltpu.VMEM((N,), jnp.float32),
            pltpu.VMEM((H,), jnp.float32),
            pltpu.SemaphoreType.DMA,
            pltpu.SemaphoreType.DMA,
            pltpu.SemaphoreType.DMA,
            pltpu.SemaphoreType.DMA,
            pltpu.SemaphoreType.DMA,
        ],
    )(A, B, row3, col3, px, py, pz, w3)


def _edge_mlp_body(pre_ref, w2_ref, be2_ref, f_ref):
    pre = pre_ref[...].astype(jnp.bfloat16)
    f_ref[...] = jnp.maximum(
        jnp.dot(pre, w2_ref[...].astype(jnp.bfloat16),
                preferred_element_type=jnp.float32)
        + be2_ref[...], 0.0)


def _edge_mlp(pre, We2, be2row):
    ne = pre.shape[0]
    be = ne // 64 if ne % 64 == 0 else BE
    assert ne % be == 0 and be % 8 == 0
    return pl.pallas_call(
        _edge_mlp_body,
        grid=(ne // be,),
        in_specs=[
            pl.BlockSpec((be, H), lambda i: (i, 0)),
            pl.BlockSpec((H, H), lambda i: (0, 0)),
            pl.BlockSpec((1, H), lambda i: (0, 0)),
        ],
        out_specs=pl.BlockSpec((be, H), lambda i: (i, 0)),
        out_shape=jax.ShapeDtypeStruct((ne, H), jnp.float32),
    )(pre, We2, be2row)


ZR = 125
RPT = N // NS


def _scatter_body(nch, f_hbm, row_hbm, out_hbm, idxb0, idxb1, fbuf0, fbuf1,
                  zbuf, aggsh, semI0, semI1, semL0, semL1):
    c = lax.axis_index("c")
    s = lax.axis_index("s")
    wid = s * NC + c
    base = wid * (nch * CB)

    def issue_load(ci, ib, fb, sI, sL):
        off = base + ci * CB
        pltpu.async_copy(row_hbm.at[pl.ds(off, CB)], ib, sI)
        pltpu.async_copy(f_hbm.at[pl.ds(off, CB)], fb, sL)

    def wait_load(ib, fb, sI, sL):
        pltpu.make_async_copy(row_hbm.at[pl.ds(base, CB)], ib, sI).wait()
        pltpu.make_async_copy(f_hbm.at[pl.ds(base, CB)], fb, sL).wait()

    def do_scatter(ib, fb):
        pltpu.sync_copy(fb, aggsh.at[ib], add=True)

    issue_load(0, idxb0, fbuf0, semI0, semL0)

    z16 = jnp.zeros((L,), jnp.float32)

    def zrow(i, carry):
        for g in range(H // L):
            zbuf[i, pl.ds(g * L, L)] = z16
        return carry

    lax.fori_loop(0, ZR, zrow, 0)
    for k in range(RPT // ZR):
        pltpu.sync_copy(zbuf, aggsh.at[pl.ds(s * RPT + k * ZR, ZR)])
    plsc.subcore_barrier()

    def pair(k, carry):
        c0 = 2 * k
        issue_load(c0 + 1, idxb1, fbuf1, semI1, semL1)
        wait_load(idxb0, fbuf0, semI0, semL0)
        do_scatter(idxb0, fbuf0)
        issue_load(c0 + 2, idxb0, fbuf0, semI0, semL0)
        wait_load(idxb1, fbuf1, semI1, semL1)
        do_scatter(idxb1, fbuf1)
        return carry

    if nch % 2 == 1:
        lax.fori_loop(0, (nch - 1) // 2, pair, 0)
        wait_load(idxb0, fbuf0, semI0, semL0)
        do_scatter(idxb0, fbuf0)
    else:
        lax.fori_loop(0, (nch - 2) // 2, pair, 0)
        issue_load(nch - 1, idxb1, fbuf1, semI1, semL1)
        wait_load(idxb0, fbuf0, semI0, semL0)
        do_scatter(idxb0, fbuf0)
        wait_load(idxb1, fbuf1, semI1, semL1)
        do_scatter(idxb1, fbuf1)

    plsc.subcore_barrier()
    pltpu.sync_copy(aggsh.at[pl.ds(s * RPT, RPT)], out_hbm.at[c, s])


def _scatter(f, row, nch):
    return pl.kernel(
        functools.partial(_scatter_body, nch),
        out_type=jax.ShapeDtypeStruct((NC, NS, RPT, H), jnp.float32),
        mesh=plsc.VectorSubcoreMesh(**_SC_MESH),
        scratch_types=[
            pltpu.VMEM((CB,), jnp.int32),
            pltpu.VMEM((CB,), jnp.int32),
            pltpu.VMEM((CB, H), jnp.float32),
            pltpu.VMEM((CB, H), jnp.float32),
            pltpu.VMEM((ZR, H), jnp.float32),
            pltpu.VMEM_SHARED((N, H), jnp.float32),
            pltpu.SemaphoreType.DMA,
            pltpu.SemaphoreType.DMA,
            pltpu.SemaphoreType.DMA,
            pltpu.SemaphoreType.DMA,
        ],
    )(f, row)


BN5 = 5000
TPB = BN5 // RPT


def _node_mlp_body(h_ref, agga_ref, aggb_ref, wn1h_ref, wn1a_ref, bn1_ref,
                   wn2_ref, bn2_ref, o_ref):
    hh = h_ref[...]
    agg = (agga_ref[0] + agga_ref[1] + aggb_ref[0] + aggb_ref[1]).reshape(BN5, H)
    t = jnp.maximum(
        jnp.dot(hh, wn1h_ref[...], preferred_element_type=jnp.float32)
        + jnp.dot(agg, wn1a_ref[...], preferred_element_type=jnp.float32)
        + bn1_ref[...], 0.0)
    o_ref[...] = hh + jnp.dot(t, wn2_ref[...], preferred_element_type=jnp.float32) \
        + bn2_ref[...]


def _node_mlp(h, aggpa, aggpb, Wn1h, Wn1a, bn1row, Wn2, bn2row):
    return pl.pallas_call(
        _node_mlp_body,
        grid=(N // BN5,),
        in_specs=[
            pl.BlockSpec((BN5, D), lambda i: (i, 0)),
            pl.BlockSpec((NC, TPB, RPT, H), lambda i: (0, i, 0, 0)),
            pl.BlockSpec((NC, TPB, RPT, H), lambda i: (0, i, 0, 0)),
            pl.BlockSpec((D, H), lambda i: (0, 0)),
            pl.BlockSpec((H, H), lambda i: (0, 0)),
            pl.BlockSpec((1, H), lambda i: (0, 0)),
            pl.BlockSpec((H, D), lambda i: (0, 0)),
            pl.BlockSpec((1, D), lambda i: (0, 0)),
        ],
        out_specs=pl.BlockSpec((BN5, D), lambda i: (i, 0)),
        out_shape=jax.ShapeDtypeStruct((N, D), jnp.float32),
    )(h, aggpa, aggpb, Wn1h, Wn1a, bn1row, Wn2, bn2row)


def kernel(h, positions, edge_index, We1, be1, We2, be2, Wn1, bn1, Wn2, bn2):
    row = edge_index[0]
    col = edge_index[1]
    row3 = row.reshape(NW, NCH, CB)
    col3 = col.reshape(NW, NCH, CB)
    row3a, row3b = row3[:, :NCHA], row3[:, NCHA:]
    col3a, col3b = col3[:, :NCHA], col3[:, NCHA:]
    rowa = row3a.reshape(-1)
    rowb = row3b.reshape(-1)
    pT = positions.T
    px, py, pz = pT[0], pT[1], pT[2]
    Wr = We1[:D]
    Wc = We1[D:2 * D]
    w3 = We1[2 * D]

    A, B = _node_tables(h, positions, Wr, Wc, w3.reshape(1, H), be1.reshape(1, H))
    be2row = be2.reshape(1, H)
    prea = _edge_pre(A, B, row3a, col3a, px, py, pz, w3, NCHA)
    preb = _edge_pre(A, B, row3b, col3b, px, py, pz, w3, NCHB)
    fa = _edge_mlp(prea, We2, be2row)
    fb = _edge_mlp(preb, We2, be2row)
    aggpa = _scatter(fa, rowa, NCHA)
    aggpb = _scatter(fb, rowb, NCHB)
    h_out = _node_mlp(h, aggpa, aggpb, Wn1[:D], Wn1[D:], bn1.reshape(1, H),
                      Wn2, bn2.reshape(1, D))
    return (h_out, positions)
